# Initial kernel scaffold; baseline (speedup 1.0000x reference)
import jax
import jax.numpy as jnp
from jax.experimental import pallas as pl


def kernel(x, gate_w, up_w, down_w):
    raise NotImplementedError("write your pallas kernel here")


# speedup vs baseline: 49.7358x; 49.7358x over previous
import functools

import jax
import jax.numpy as jnp
from jax.experimental import pallas as pl

_K = 512


def _moc_block(x_ref, gw_ref, uw_ref, dw_ref, o_ref, *, k):
    x = x_ref[...]
    gate = jax.lax.dot_general(x, gw_ref[...], (((1,), (1,)), ((), ())),
                               preferred_element_type=jnp.float32)
    v = jax.lax.dot_general(x, uw_ref[...], (((1,), (1,)), ((), ())),
                            preferred_element_type=jnp.float32)

    bits = jax.lax.bitcast_convert_type(gate, jnp.int32)
    keys = jnp.where(bits < 0, bits ^ jnp.int32(0x7FFFFFFF), bits)

    cnt0 = jnp.sum((keys >= 0).astype(jnp.int32), axis=1)
    t = jnp.where(cnt0 >= k, jnp.int32(0), jnp.iinfo(jnp.int32).min)

    def body(i, t):
        cand = t + (jnp.int32(1) << (30 - i))
        cnt = jnp.sum((keys >= cand[:, None]).astype(jnp.int32), axis=1)
        return jnp.where(cnt >= k, cand, t)

    t = jax.lax.fori_loop(0, 31, body, t)

    act = gate * jax.nn.sigmoid(gate) * v
    act = jnp.where(keys >= t[:, None], act, 0.0)
    o_ref[...] = jax.lax.dot_general(act, dw_ref[...], (((1,), (1,)), ((), ())),
                                     preferred_element_type=jnp.float32)


@jax.jit
def kernel(x, gate_w, up_w, down_w):
    B, S, H = x.shape
    I = gate_w.shape[0]
    rows = B * S
    R = 256
    x2 = x.reshape(rows, H)
    out = pl.pallas_call(
        functools.partial(_moc_block, k=min(_K, I)),
        grid=(rows // R,),
        in_specs=[
            pl.BlockSpec((R, H), lambda i: (i, 0)),
            pl.BlockSpec((I, H), lambda i: (0, 0)),
            pl.BlockSpec((I, H), lambda i: (0, 0)),
            pl.BlockSpec((H, I), lambda i: (0, 0)),
        ],
        out_specs=pl.BlockSpec((R, H), lambda i: (i, 0)),
        out_shape=jax.ShapeDtypeStruct((rows, H), jnp.float32),
    )(x2, gate_w, up_w, down_w)
    return out.reshape(B, S, H)
